# baseline (speedup 1.0000x reference)
import jax
import jax.numpy as jnp
from jax import lax
from jax.experimental import pallas as pl
from jax.experimental.pallas import tpu as pltpu
from jax.experimental.pallas import tpu_sc as plsc

_SEQ = 200
_D = 64
_L = 16
_NC = 2
_NS = 16
_NW = _NC * _NS
_SCALE = 8.0


def _body(idx_hbm, table_hbm, pos_hbm, out_hbm, idx_v, rows_v, pos_v, gsem):
    wid = lax.axis_index("s") * _NC + lax.axis_index("c")
    batches_per_worker = idx_hbm.shape[0] // (_SEQ * _NW)

    pltpu.sync_copy(pos_hbm, pos_v)

    def chunk(g, carry):
        base = (wid * batches_per_worker + g) * _SEQ
        pltpu.sync_copy(idx_hbm.at[pl.ds(base, _SEQ)], idx_v)
        pltpu.async_copy(table_hbm.at[idx_v], rows_v, gsem).wait()

        def row(r, c):
            for dd in range(_D // _L):
                sl = pl.ds(dd * _L, _L)
                rows_v[r, sl] = rows_v[r, sl] * _SCALE + pos_v[r, sl]
            return c

        lax.fori_loop(0, _SEQ, row, 0)
        pltpu.sync_copy(rows_v, out_hbm.at[pl.ds(base, _SEQ)])
        return carry

    lax.fori_loop(0, batches_per_worker, chunk, 0)


def kernel(inputs, token_table, pos_table):
    b, s = inputs.shape
    _, d = token_table.shape
    idx = inputs.reshape(-1)
    mesh = plsc.VectorSubcoreMesh(
        core_axis_name="c", subcore_axis_name="s",
        num_cores=_NC, num_subcores=_NS,
    )
    out = pl.kernel(
        _body,
        out_type=jax.ShapeDtypeStruct((b * s, d), jnp.float32),
        mesh=mesh,
        compiler_params=pltpu.CompilerParams(use_tc_tiling_on_sc=False),
        scratch_types=[
            pltpu.VMEM((_SEQ,), jnp.int32),
            pltpu.VMEM((_SEQ, _D), jnp.float32),
            pltpu.VMEM((_SEQ, _D), jnp.float32),
            pltpu.SemaphoreType.DMA,
        ],
    )(idx, token_table, pos_table)
    return out.reshape(b, s, d)


# speedup vs baseline: 1.1873x; 1.1873x over previous
import jax
import jax.numpy as jnp
from jax import lax
from jax.experimental import pallas as pl
from jax.experimental.pallas import tpu as pltpu
from jax.experimental.pallas import tpu_sc as plsc

_SEQ = 200
_D = 64
_L = 16
_NC = 2
_NS = 16
_NW = _NC * _NS
_SCALE = 8.0


def _body(idx_hbm, table_hbm, pos_hbm, out_hbm, idx_v, rows_v, pos_v,
          isem0, isem1, gsem0, gsem1, osem0, osem1):
    isem = (isem0, isem1)
    gsem = (gsem0, gsem1)
    osem = (osem0, osem1)
    wid = lax.axis_index("s") * _NC + lax.axis_index("c")
    num_chunks = idx_hbm.shape[0] // (_SEQ * _NW)
    wbase = wid * num_chunks

    def idx_start(slot, g):
        base = (wbase + g) * _SEQ
        pltpu.async_copy(idx_hbm.at[pl.ds(base, _SEQ)], idx_v.at[slot],
                         isem[slot])

    def idx_wait(slot):
        pltpu.make_async_copy(idx_hbm.at[pl.ds(0, _SEQ)], idx_v.at[slot],
                              isem[slot]).wait()

    def gather_start(slot):
        pltpu.async_copy(table_hbm.at[idx_v.at[slot]], rows_v.at[slot],
                         gsem[slot])

    def gather_wait(slot):
        pltpu.make_async_copy(table_hbm.at[idx_v.at[slot]], rows_v.at[slot],
                              gsem[slot]).wait()

    def out_start(slot, g):
        base = (wbase + g) * _SEQ
        pltpu.async_copy(rows_v.at[slot], out_hbm.at[pl.ds(base, _SEQ)],
                         osem[slot])

    def out_wait(slot):
        pltpu.make_async_copy(rows_v.at[slot], out_hbm.at[pl.ds(0, _SEQ)],
                              osem[slot]).wait()

    pltpu.sync_copy(pos_hbm, pos_v)

    idx_start(0, 0)
    idx_wait(0)
    gather_start(0)
    idx_start(1, 1)

    def pair(p, carry):
        for b in (0, 1):
            g = 2 * p + b
            nb = 1 - b

            @pl.when(g + 1 < num_chunks)
            def _():
                idx_wait(nb)

            @pl.when(g >= 1)
            def _():
                out_wait(nb)

            @pl.when(g + 1 < num_chunks)
            def _():
                gather_start(nb)

            gather_wait(b)

            @pl.when(g + 2 < num_chunks)
            def _():
                idx_start(b, g + 2)

            rows = rows_v.at[b]

            @plsc.parallel_loop(0, _SEQ, step=1, unroll=4)
            def _(r):
                for dd in range(_D // _L):
                    sl = pl.ds(dd * _L, _L)
                    rows[r, sl] = rows[r, sl] * _SCALE + pos_v[r, sl]

            out_start(b, g)
        return carry

    lax.fori_loop(0, num_chunks // 2, pair, 0)
    out_wait((num_chunks - 1) % 2)


def kernel(inputs, token_table, pos_table):
    b, s = inputs.shape
    _, d = token_table.shape
    idx = inputs.reshape(-1)
    mesh = plsc.VectorSubcoreMesh(
        core_axis_name="c", subcore_axis_name="s",
        num_cores=_NC, num_subcores=_NS,
    )
    out = pl.kernel(
        _body,
        out_type=jax.ShapeDtypeStruct((b * s, d), jnp.float32),
        mesh=mesh,
        compiler_params=pltpu.CompilerParams(use_tc_tiling_on_sc=False),
        scratch_types=[
            pltpu.VMEM((2, _SEQ), jnp.int32),
            pltpu.VMEM((2, _SEQ, _D), jnp.float32),
            pltpu.VMEM((_SEQ, _D), jnp.float32),
        ] + [pltpu.SemaphoreType.DMA] * 6,
    )(idx, token_table, pos_table)
    return out.reshape(b, s, d)
